# baseline (speedup 1.0000x reference)
import functools

import jax
import jax.numpy as jnp
from jax import lax
from jax.experimental import pallas as pl
from jax.experimental.pallas import tpu as pltpu
from jax.experimental.pallas import tpu_sc as plsc

B = 4
T = 4096
VOCAB = 256
BYTE_DIM = 32
NUM_SCALES = 8
EMBED_PER_SCALE = 16
BUCKETS = 65536
COMBO_BUCKETS = 65536
COMBO_DIM = 16
CONV_K = 4
MATCH_OFFSETS = (1, 2, 4, 8)
SCAN_DIM = 32
HASH_DIM = 80
FEAT_DIM = 116

def _pow_consts():
    p = 1000003
    out = []
    for _ in range(1, NUM_SCALES):
        out.append(p if p < 2**31 else p - 2**32)
        p = (p * p) % (2**32)
    return tuple(out)

_P_POWS = _pow_consts()

NW = 32
PYR_IDX = NUM_SCALES * B * T
CHUNK1 = PYR_IDX // NW
CHUNK2 = (B * T) // NW
IDXW = 128
N1 = CHUNK1 // IDXW
N2 = CHUNK2 // IDXW


def _idx_body(chars_ref, idx1_ref, idx2_ref):
    h = chars_ref[...]
    combo = h
    idx1_ref[0] = h & (BUCKETS - 1)
    for s in range(1, NUM_SCALES):
        w = 1 << (s - 1)
        zpad = jnp.zeros((B, w), jnp.int32)
        shifted = jnp.concatenate([zpad, h[:, : T - w]], axis=1)
        h = h + _P_POWS[s - 1] * shifted
        combo = combo ^ h
        idx1_ref[s] = h & (BUCKETS - 1)
    idx2_ref[0] = combo & (COMBO_BUCKETS - 1)


_IDX_SPECS = dict(
    out_shape=(
        jax.ShapeDtypeStruct((NUM_SCALES, B, T), jnp.int32),
        jax.ShapeDtypeStruct((1, B, T), jnp.int32),
    ),
)


def _sc_gather(pyr_tables, combo_table, idx1, idx2):
    mesh = plsc.VectorSubcoreMesh(core_axis_name="c", subcore_axis_name="s")

    @functools.partial(
        pl.kernel,
        mesh=mesh,
        out_type=(
            jax.ShapeDtypeStruct((NUM_SCALES, B, T, EMBED_PER_SCALE),
                                 jnp.float32),
            jax.ShapeDtypeStruct((B, T, COMBO_DIM), jnp.float32),
        ),
        scratch_types=[
            pltpu.VMEM((N1, IDXW), jnp.int32),
            pltpu.VMEM((CHUNK1, EMBED_PER_SCALE), jnp.float32),
            pltpu.VMEM((N2, IDXW), jnp.int32),
            pltpu.VMEM((CHUNK2, COMBO_DIM), jnp.float32),
            pltpu.SemaphoreType.DMA,
        ],
        compiler_params=pltpu.CompilerParams(use_tc_tiling_on_sc=False),
    )
    def k(pyr_hbm, combo_hbm, idx1_hbm, idx2_hbm, out1_hbm, out2_hbm,
          idx1_v, rows1_v, idx2_v, rows2_v, sem):
        wid = lax.axis_index("s") * 2 + lax.axis_index("c")
        s_id = wid // 4
        b_id = wid % 4
        pltpu.sync_copy(idx1_hbm.at[wid], idx1_v)
        pltpu.sync_copy(idx2_hbm.at[wid], idx2_v)
        table = pyr_hbm.at[s_id]

        @pl.loop(0, N1)
        def _(j):
            pltpu.async_copy(
                table.at[idx1_v.at[j]],
                rows1_v.at[pl.ds(j * IDXW, IDXW)], sem
            ).wait()

        @pl.loop(0, N2)
        def _(j):
            pltpu.async_copy(
                combo_hbm.at[idx2_v.at[j]],
                rows2_v.at[pl.ds(j * IDXW, IDXW)], sem
            ).wait()

        pltpu.sync_copy(rows1_v, out1_hbm.at[s_id].at[b_id])
        pltpu.sync_copy(
            rows2_v,
            out2_hbm.at[wid // 8].at[pl.ds((wid % 8) * CHUNK2, CHUNK2)])

    return k(pyr_tables, combo_table, idx1, idx2)


def _sig(x):
    return 1.0 / (1.0 + jnp.exp(-x))


def _dense_body(chars_ref, g1_ref, g2_ref, byte_embed_ref, gate_bias_ref,
                convw_ref, convb_ref, scan_win_ref, scan_a_ref,
                scan_wout_ref, out_w_ref, out_b_ref, out_ref):
    ch = chars_ref[0]

    vocab_iota = lax.broadcasted_iota(jnp.int32, (T, VOCAB), 1)
    onehot = (ch == vocab_iota).astype(jnp.float32)
    byte_emb = jnp.dot(onehot, byte_embed_ref[...],
                       preferred_element_type=jnp.float32)

    gb_all = gate_bias_ref[...]
    parts = []
    for i in range(NUM_SCALES // 2):
        val = g1_ref[2 * i, 0]
        gate = g1_ref[2 * i + 1, 0]
        parts.append(val * _sig(gate + gb_all[i:i + 1]))
    hf = jnp.concatenate(parts + [g2_ref[0]], axis=1)

    w_all = convw_ref[...]
    acc = hf * w_all[CONV_K - 1:CONV_K]
    for j in range(CONV_K - 1):
        m = CONV_K - 1 - j
        shifted = jnp.concatenate(
            [jnp.zeros((m, HASH_DIM), jnp.float32), hf[: T - m]], axis=0)
        acc = acc + shifted * w_all[j:j + 1]
    acc = acc + convb_ref[...]
    hf2 = acc * _sig(acc)

    mfs = []
    for k in MATCH_OFFSETS:
        shifted = jnp.concatenate(
            [jnp.full((k, 1), -1, jnp.int32), ch[: T - k]], axis=0)
        mfs.append((ch == shifted).astype(jnp.float32))
    mf = jnp.concatenate(mfs, axis=1)

    h = jnp.concatenate([byte_emb, hf2, mf], axis=1)

    u = jnp.dot(h, scan_win_ref[...], preferred_element_type=jnp.float32)
    d = _sig(scan_a_ref[...])
    x = (1.0 - d) * u
    dp = d
    sh = 1
    while sh < T:
        shifted = jnp.concatenate(
            [jnp.zeros((sh, SCAN_DIM), jnp.float32), x[: T - sh]], axis=0)
        x = x + dp * shifted
        dp = dp * dp
        sh *= 2
    hs = x

    h2 = h + jnp.dot(hs, scan_wout_ref[...], preferred_element_type=jnp.float32)
    out = jnp.dot(h2, out_w_ref[...], preferred_element_type=jnp.float32)
    out_ref[0] = out + out_b_ref[...]


_DENSE_SPECS = dict(
    grid=(B,),
    in_specs=[
        pl.BlockSpec((1, T, 1), lambda b: (b, 0, 0)),
        pl.BlockSpec((NUM_SCALES, 1, T, EMBED_PER_SCALE),
                     lambda b: (0, b, 0, 0)),
        pl.BlockSpec((1, T, COMBO_DIM), lambda b: (b, 0, 0)),
        pl.BlockSpec((VOCAB, BYTE_DIM), lambda b: (0, 0)),
        pl.BlockSpec((NUM_SCALES // 2, EMBED_PER_SCALE), lambda b: (0, 0)),
        pl.BlockSpec((CONV_K, HASH_DIM), lambda b: (0, 0)),
        pl.BlockSpec((1, HASH_DIM), lambda b: (0, 0)),
        pl.BlockSpec((FEAT_DIM, SCAN_DIM), lambda b: (0, 0)),
        pl.BlockSpec((1, SCAN_DIM), lambda b: (0, 0)),
        pl.BlockSpec((SCAN_DIM, FEAT_DIM), lambda b: (0, 0)),
        pl.BlockSpec((FEAT_DIM, VOCAB), lambda b: (0, 0)),
        pl.BlockSpec((1, VOCAB), lambda b: (0, 0)),
    ],
    out_specs=pl.BlockSpec((1, T, VOCAB), lambda b: (b, 0, 0)),
    out_shape=jax.ShapeDtypeStruct((B, T, VOCAB), jnp.float32),
    compiler_params=pltpu.CompilerParams(
        dimension_semantics=("parallel",)),
)


def kernel(chars, byte_embed, pyr_tables, combo_table, gate_bias, conv_w,
           conv_b, scan_win, scan_a, scan_wout, out_w, out_b):
    idx1, idx2 = pl.pallas_call(_idx_body, **_IDX_SPECS)(chars)
    g1, g2 = _sc_gather(
        pyr_tables,
        combo_table,
        idx1.reshape(NW, N1, IDXW),
        idx2.reshape(NW, N2, IDXW),
    )
    out = pl.pallas_call(_dense_body, **_DENSE_SPECS)(
        chars.reshape(B, T, 1),
        g1,
        g2,
        byte_embed,
        gate_bias,
        jnp.transpose(conv_w[:, 0, :]),
        conv_b.reshape(1, HASH_DIM),
        scan_win,
        scan_a.reshape(1, SCAN_DIM),
        scan_wout,
        out_w,
        out_b.reshape(1, VOCAB),
    )
    return out


# speedup vs baseline: 1.1277x; 1.1277x over previous
import functools

import jax
import jax.numpy as jnp
from jax import lax
from jax.experimental import pallas as pl
from jax.experimental.pallas import tpu as pltpu
from jax.experimental.pallas import tpu_sc as plsc

B = 4
T = 4096
VOCAB = 256
BYTE_DIM = 32
NUM_SCALES = 8
EMBED_PER_SCALE = 16
BUCKETS = 65536
COMBO_BUCKETS = 65536
COMBO_DIM = 16
CONV_K = 4
MATCH_OFFSETS = (1, 2, 4, 8)
SCAN_DIM = 32
HASH_DIM = 80
FEAT_DIM = 116

TR = T // 128

def _pow_consts():
    p = 1000003
    out = []
    for _ in range(1, NUM_SCALES):
        out.append(p if p < 2**31 else p - 2**32)
        p = (p * p) % (2**32)
    return tuple(out)

_P_POWS = _pow_consts()

NW = 32
CHUNK1 = T
CHUNK2 = (B * T) // NW
IDXW = 128
N1 = CHUNK1 // IDXW
N2 = CHUNK2 // IDXW


def _shift_tok(x, w):
    down = jnp.concatenate(
        [jnp.zeros((B, 1, 128), x.dtype), x[:, : TR - 1]], axis=1)
    return jnp.concatenate([down[:, :, 128 - w:], x[:, :, : 128 - w]], axis=2)


def _idx_body(chars_ref, idx1_ref, idx2_ref):
    h = chars_ref[...]
    combo = h
    idx1_ref[0] = h & (BUCKETS - 1)
    for s in range(1, NUM_SCALES):
        w = 1 << (s - 1)
        h = h + _P_POWS[s - 1] * _shift_tok(h, w)
        combo = combo ^ h
        idx1_ref[s] = h & (BUCKETS - 1)
    idx2_ref[...] = combo & (COMBO_BUCKETS - 1)


_IDX_SPECS = dict(
    out_shape=(
        jax.ShapeDtypeStruct((NUM_SCALES, B, TR, 128), jnp.int32),
        jax.ShapeDtypeStruct((B, TR, 128), jnp.int32),
    ),
)


def _sc_gather(pyr_tables, combo_table, idx1, idx2):
    mesh = plsc.VectorSubcoreMesh(core_axis_name="c", subcore_axis_name="s")

    @functools.partial(
        pl.kernel,
        mesh=mesh,
        out_type=(
            jax.ShapeDtypeStruct((B, T, NUM_SCALES * EMBED_PER_SCALE),
                                 jnp.float32),
            jax.ShapeDtypeStruct((B, T, COMBO_DIM), jnp.float32),
        ),
        scratch_types=[
            pltpu.VMEM((N1, IDXW), jnp.int32),
            pltpu.VMEM((CHUNK1, EMBED_PER_SCALE), jnp.float32),
            pltpu.VMEM((N2, IDXW), jnp.int32),
            pltpu.VMEM((CHUNK2, COMBO_DIM), jnp.float32),
            pltpu.SemaphoreType.DMA,
        ],
        compiler_params=pltpu.CompilerParams(use_tc_tiling_on_sc=False),
    )
    def k(pyr_hbm, combo_hbm, idx1_hbm, idx2_hbm, out1_hbm, out2_hbm,
          idx1_v, rows1_v, idx2_v, rows2_v, sem):
        wid = lax.axis_index("s") * 2 + lax.axis_index("c")
        s_id = wid // 4
        b_id = wid % 4
        pltpu.sync_copy(idx1_hbm.at[wid], idx1_v)
        pltpu.sync_copy(idx2_hbm.at[wid], idx2_v)
        table = pyr_hbm.at[s_id]

        @pl.loop(0, N1)
        def _(j):
            pltpu.async_copy(
                table.at[idx1_v.at[j]],
                rows1_v.at[pl.ds(j * IDXW, IDXW)], sem
            ).wait()

        @pl.loop(0, N2)
        def _(j):
            pltpu.async_copy(
                combo_hbm.at[idx2_v.at[j]],
                rows2_v.at[pl.ds(j * IDXW, IDXW)], sem
            ).wait()

        pltpu.sync_copy(
            rows1_v,
            out1_hbm.at[b_id, :, pl.ds(s_id * EMBED_PER_SCALE,
                                       EMBED_PER_SCALE)])
        pltpu.sync_copy(
            rows2_v,
            out2_hbm.at[wid // 8].at[pl.ds((wid % 8) * CHUNK2, CHUNK2)])

    return k(pyr_tables, combo_table, idx1, idx2)


def _sig(x):
    return 1.0 / (1.0 + jnp.exp(-x))


def _dense_body(chars_ref, g1_ref, g2_ref, byte_embedT_ref, gate_bias_ref,
                convw_ref, convb_ref, scan_win_ref, scan_a_ref,
                scan_wout_ref, out_w_ref, out_b_ref, out_ref):
    ch2 = chars_ref[pl.ds(pl.program_id(0), 1), :]

    vocab_iota = lax.broadcasted_iota(jnp.int32, (VOCAB, T), 0)
    onehotT = (ch2 == vocab_iota).astype(jnp.float32)
    byte_emb = jnp.transpose(
        jnp.dot(byte_embedT_ref[...], onehotT,
                preferred_element_type=jnp.float32))

    gp = g1_ref[0]
    gb_all = gate_bias_ref[...]
    parts = []
    for i in range(NUM_SCALES // 2):
        val = gp[:, 32 * i: 32 * i + 16]
        gate = gp[:, 32 * i + 16: 32 * i + 32]
        parts.append(val * _sig(gate + gb_all[i:i + 1]))
    hf = jnp.concatenate(parts + [g2_ref[0]], axis=1)

    w_all = convw_ref[...]
    acc = hf * w_all[CONV_K - 1:CONV_K]
    for j in range(CONV_K - 1):
        m = CONV_K - 1 - j
        shifted = jnp.concatenate(
            [jnp.zeros((m, HASH_DIM), jnp.float32), hf[: T - m]], axis=0)
        acc = acc + shifted * w_all[j:j + 1]
    acc = acc + convb_ref[...]
    hf2 = acc * _sig(acc)

    mfs = []
    for k in MATCH_OFFSETS:
        shifted = jnp.concatenate(
            [jnp.full((1, k), -1, jnp.int32), ch2[:, : T - k]], axis=1)
        mfs.append((ch2 == shifted).astype(jnp.float32))
    mf = jnp.transpose(jnp.concatenate(mfs, axis=0))

    h = jnp.concatenate([byte_emb, hf2, mf], axis=1)

    u = jnp.dot(h, scan_win_ref[...], preferred_element_type=jnp.float32)
    d = _sig(scan_a_ref[...])
    x = (1.0 - d) * u
    dp = d
    sh = 1
    while sh < T:
        shifted = jnp.concatenate(
            [jnp.zeros((sh, SCAN_DIM), jnp.float32), x[: T - sh]], axis=0)
        x = x + dp * shifted
        dp = dp * dp
        sh *= 2
    hs = x

    h2 = h + jnp.dot(hs, scan_wout_ref[...], preferred_element_type=jnp.float32)
    out = jnp.dot(h2, out_w_ref[...], preferred_element_type=jnp.float32)
    out_ref[0] = out + out_b_ref[...]


_DENSE_SPECS = dict(
    grid=(B,),
    in_specs=[
        pl.BlockSpec((B, T), lambda b: (0, 0)),
        pl.BlockSpec((1, T, NUM_SCALES * EMBED_PER_SCALE),
                     lambda b: (b, 0, 0)),
        pl.BlockSpec((1, T, COMBO_DIM), lambda b: (b, 0, 0)),
        pl.BlockSpec((BYTE_DIM, VOCAB), lambda b: (0, 0)),
        pl.BlockSpec((NUM_SCALES // 2, EMBED_PER_SCALE), lambda b: (0, 0)),
        pl.BlockSpec((CONV_K, HASH_DIM), lambda b: (0, 0)),
        pl.BlockSpec((1, HASH_DIM), lambda b: (0, 0)),
        pl.BlockSpec((FEAT_DIM, SCAN_DIM), lambda b: (0, 0)),
        pl.BlockSpec((1, SCAN_DIM), lambda b: (0, 0)),
        pl.BlockSpec((SCAN_DIM, FEAT_DIM), lambda b: (0, 0)),
        pl.BlockSpec((FEAT_DIM, VOCAB), lambda b: (0, 0)),
        pl.BlockSpec((1, VOCAB), lambda b: (0, 0)),
    ],
    out_specs=pl.BlockSpec((1, T, VOCAB), lambda b: (b, 0, 0)),
    out_shape=jax.ShapeDtypeStruct((B, T, VOCAB), jnp.float32),
    compiler_params=pltpu.CompilerParams(
        dimension_semantics=("parallel",)),
)


def kernel(chars, byte_embed, pyr_tables, combo_table, gate_bias, conv_w,
           conv_b, scan_win, scan_a, scan_wout, out_w, out_b):
    chars3 = chars.reshape(B, TR, 128)
    idx1, idx2 = pl.pallas_call(_idx_body, **_IDX_SPECS)(chars3)
    g1, g2 = _sc_gather(
        pyr_tables,
        combo_table,
        idx1.reshape(NW, N1, IDXW),
        idx2.reshape(NW, N2, IDXW),
    )
    out = pl.pallas_call(_dense_body, **_DENSE_SPECS)(
        chars,
        g1,
        g2,
        jnp.transpose(byte_embed),
        gate_bias,
        jnp.transpose(conv_w[:, 0, :]),
        conv_b.reshape(1, HASH_DIM),
        scan_win,
        scan_a.reshape(1, SCAN_DIM),
        scan_wout,
        out_w,
        out_b.reshape(1, VOCAB),
    )
    return out
